# baseline (speedup 1.0000x reference)
import jax
import jax.numpy as jnp
from jax import lax
from jax.experimental import pallas as pl
from jax.experimental.pallas import tpu as pltpu
from jax.experimental.pallas import tpu_sc as plsc

H = 128
N_NODES = 50000
E_EDGES = 800000
_NW = 32
_C = 3200
_NCHUNK = E_EDGES // _C
_GC = 128
_GNCH = 6400
_E_PAD_G = _GNCH * _GC
_GLOC = _GNCH // _NW
_NODE_BLK = 2000
_FIN_BLK = 2048
_N_PAD = 51200
_EDGE_BLK = 3200


def _segsum_body(vt_hbm, dst_hbm, out_hbm,
                 d_a, v0_a, v1_a, d_b, v0_b, v1_b,
                 acc0, acc1, sem_a, sem_b):
    wid = lax.axis_index("s") * 2 + lax.axis_index("c")

    def start(chunk, f0, bufs, sem):
        d_buf, v0_buf, v1_buf = bufs
        pltpu.async_copy(dst_hbm.at[pl.ds(chunk * _C, _C)], d_buf, sem)
        pltpu.async_copy(vt_hbm.at[f0, pl.ds(chunk * _C, _C)], v0_buf, sem)
        pltpu.async_copy(vt_hbm.at[f0 + 1, pl.ds(chunk * _C, _C)], v1_buf, sem)

    def wait(bufs, sem):
        d_buf, v0_buf, v1_buf = bufs
        pltpu.make_async_copy(dst_hbm.at[pl.ds(0, _C)], d_buf, sem).wait()
        pltpu.make_async_copy(vt_hbm.at[0, pl.ds(0, _C)], v0_buf, sem).wait()
        pltpu.make_async_copy(vt_hbm.at[0, pl.ds(0, _C)], v1_buf, sem).wait()

    def consume(bufs):
        d_buf, v0_buf, v1_buf = bufs

        def inner(j, _):
            d = d_buf[pl.ds(j * 16, 16)]
            plsc.addupdate_scatter(acc0, [d], v0_buf[pl.ds(j * 16, 16)])
            plsc.addupdate_scatter(acc1, [d], v1_buf[pl.ds(j * 16, 16)])
            return 0

        lax.fori_loop(0, _C // 16, inner, 0)

    bufs_a = (d_a, v0_a, v1_a)
    bufs_b = (d_b, v0_b, v1_b)
    zeros16 = jnp.zeros((16,), jnp.float32)

    for p in range(2):
        f0 = (p * _NW + wid) * 2

        def zero(i, _):
            acc0[pl.ds(i * 16, 16)] = zeros16
            acc1[pl.ds(i * 16, 16)] = zeros16
            return 0

        lax.fori_loop(0, N_NODES // 16, zero, 0)

        start(0, f0, bufs_a, sem_a)

        def body2(h, _):
            c0 = h * 2
            start(c0 + 1, f0, bufs_b, sem_b)
            wait(bufs_a, sem_a)
            consume(bufs_a)

            @pl.when(c0 + 2 < _NCHUNK)
            def _():
                start(c0 + 2, f0, bufs_a, sem_a)

            wait(bufs_b, sem_b)
            consume(bufs_b)
            return 0

        lax.fori_loop(0, _NCHUNK // 2, body2, 0)

        pltpu.sync_copy(acc0, out_hbm.at[f0])
        pltpu.sync_copy(acc1, out_hbm.at[f0 + 1])


def _sc_segsum_t(vt, dst):
    mesh = plsc.VectorSubcoreMesh(core_axis_name="c", subcore_axis_name="s")
    return pl.kernel(
        _segsum_body,
        mesh=mesh,
        compiler_params=pltpu.CompilerParams(needs_layout_passes=False),
        out_type=jax.ShapeDtypeStruct((H, N_NODES), jnp.float32),
        scratch_types=[
            pltpu.VMEM((_C,), jnp.int32),
            pltpu.VMEM((_C,), jnp.float32),
            pltpu.VMEM((_C,), jnp.float32),
            pltpu.VMEM((_C,), jnp.int32),
            pltpu.VMEM((_C,), jnp.float32),
            pltpu.VMEM((_C,), jnp.float32),
            pltpu.VMEM((N_NODES,), jnp.float32),
            pltpu.VMEM((N_NODES,), jnp.float32),
            pltpu.SemaphoreType.DMA,
            pltpu.SemaphoreType.DMA,
        ],
    )(vt, dst)


def _gather2_body(a_hbm, b_hbm, src2_hbm, dst2_hbm, oa_hbm, ob_hbm,
                  ia, ib, arows, brows, sem):
    wid = lax.axis_index("s") * 2 + lax.axis_index("c")
    lo = wid * _GLOC

    def chunk(c, _):
        pltpu.sync_copy(src2_hbm.at[c], ia)
        pltpu.sync_copy(dst2_hbm.at[c], ib)
        cp_a = pltpu.async_copy(a_hbm.at[ia], arows, sem)
        cp_b = pltpu.async_copy(b_hbm.at[ib], brows, sem)
        cp_a.wait()
        cp_b.wait()
        pltpu.sync_copy(arows, oa_hbm.at[pl.ds(c * _GC, _GC)])
        pltpu.sync_copy(brows, ob_hbm.at[pl.ds(c * _GC, _GC)])
        return 0

    lax.fori_loop(lo, lo + _GLOC, chunk, 0)


def _sc_gather2(a_tab, b_tab, src, dst):
    mesh = plsc.VectorSubcoreMesh(core_axis_name="c", subcore_axis_name="s")
    pad = _E_PAD_G - E_EDGES
    src2 = jnp.pad(src, (0, pad)).reshape(_GNCH, _GC)
    dst2 = jnp.pad(dst, (0, pad)).reshape(_GNCH, _GC)
    return pl.kernel(
        _gather2_body,
        mesh=mesh,
        compiler_params=pltpu.CompilerParams(needs_layout_passes=False),
        out_type=(jax.ShapeDtypeStruct((_E_PAD_G, H), jnp.float32),
                  jax.ShapeDtypeStruct((_E_PAD_G, H), jnp.float32)),
        scratch_types=[
            pltpu.VMEM((_GC,), jnp.int32),
            pltpu.VMEM((_GC,), jnp.int32),
            pltpu.VMEM((_GC, H), jnp.float32),
            pltpu.VMEM((_GC, H), jnp.float32),
            pltpu.SemaphoreType.DMA,
        ],
    )(a_tab, b_tab, src2, dst2)


def _node_pre_body(nf, wn, bn, wl, bl, w1, wr, br, w2, nn1_o, a_o, b_o):
    x = nf[...]
    nn1_o[...] = jnp.maximum(
        jnp.dot(x, wn[...], preferred_element_type=jnp.float32) + bn[...], 0.0)
    left = jnp.maximum(
        jnp.dot(x, wl[...], preferred_element_type=jnp.float32) + bl[...], 0.0)
    a_o[...] = jnp.dot(left, w1[...], preferred_element_type=jnp.float32)
    right = jnp.maximum(
        jnp.dot(x, wr[...], preferred_element_type=jnp.float32) + br[...], 0.0)
    b_o[...] = jnp.dot(right, w2[...], preferred_element_type=jnp.float32)


def _node_pre(nf, wn, bn, wl, bl, w1, wr, br, w2):
    n = nf.shape[0]
    blk = _NODE_BLK
    full = lambda s: pl.BlockSpec(s, lambda i: (0,) * len(s))
    return pl.pallas_call(
        _node_pre_body,
        grid=(n // blk,),
        in_specs=[
            pl.BlockSpec((blk, nf.shape[1]), lambda i: (i, 0)),
            full(wn.shape), full((1, H)), full(wl.shape), full((1, H)),
            full(w1.shape), full(wr.shape), full((1, H)), full(w2.shape),
        ],
        out_specs=[pl.BlockSpec((blk, H), lambda i: (i, 0))] * 3,
        out_shape=[jax.ShapeDtypeStruct((n, H), jnp.float32)] * 3,
    )(nf, wn, bn.reshape(1, H), wl, bl.reshape(1, H), w1, wr,
      br.reshape(1, H), w2)


def _e2nT_body(ef, we2n, bcol, out_t):
    out_t[...] = jnp.maximum(
        lax.dot_general(we2n[...], ef[...], (((0,), (1,)), ((), ())),
                        preferred_element_type=jnp.float32) + bcol[...], 0.0)


def _e2n_t(ef8, we2n8, be2n):
    blk = _EDGE_BLK
    full = lambda s: pl.BlockSpec(s, lambda i: (0,) * len(s))
    return pl.pallas_call(
        _e2nT_body,
        grid=(E_EDGES // blk,),
        in_specs=[
            pl.BlockSpec((blk, 8), lambda i: (i, 0)),
            full((8, H)), full((H, 1)),
        ],
        out_specs=pl.BlockSpec((H, blk), lambda i: (0, i)),
        out_shape=jax.ShapeDtypeStruct((H, E_EDGES), jnp.float32),
    )(ef8, we2n8, be2n.reshape(H, 1))


def _edge2_body(ef, asrc, bdst, we2e, be2e, w3, bu, wl2, bl2col, out_t):
    x = ef[...]
    e2e = jnp.maximum(
        jnp.dot(x, we2e[...], preferred_element_type=jnp.float32) + be2e[...], 0.0)
    t = jnp.dot(e2e, w3[...], preferred_element_type=jnp.float32)
    new_e = jnp.maximum(asrc[...] + bdst[...] + t + bu[...], 0.0)
    out_t[...] = jnp.maximum(
        lax.dot_general(wl2[...], new_e, (((0,), (1,)), ((), ())),
                        preferred_element_type=jnp.float32) + bl2col[...], 0.0)


def _edge2_t(ef8, asrc, bdst, we2e8, be2e, w3, bu, wl2, bl2):
    blk = _EDGE_BLK
    full = lambda s: pl.BlockSpec(s, lambda i: (0,) * len(s))
    return pl.pallas_call(
        _edge2_body,
        grid=(E_EDGES // blk,),
        in_specs=[
            pl.BlockSpec((blk, 8), lambda i: (i, 0)),
            pl.BlockSpec((blk, H), lambda i: (i, 0)),
            pl.BlockSpec((blk, H), lambda i: (i, 0)),
            full((8, H)), full((1, H)), full((H, H)), full((1, H)),
            full((H, H)), full((H, 1)),
        ],
        out_specs=pl.BlockSpec((H, blk), lambda i: (0, i)),
        out_shape=jax.ShapeDtypeStruct((H, E_EDGES), jnp.float32),
    )(ef8, asrc, bdst, we2e8, be2e.reshape(1, H), w3, bu.reshape(1, H),
      wl2, bl2.reshape(H, 1))


def _node_fin_body(nn1, agg_t, agg2_t, u1, u2, bu, w, b2, v1, v2, b3, wg, bg,
                   hh_o):
    agg_term = lax.dot_general(agg_t[...], u2[...], (((0,), (0,)), ((), ())),
                               preferred_element_type=jnp.float32)
    new_n = jnp.maximum(
        jnp.dot(nn1[...], u1[...], preferred_element_type=jnp.float32)
        + agg_term + bu[...], 0.0)
    nn2 = jnp.maximum(
        jnp.dot(new_n, w[...], preferred_element_type=jnp.float32) + b2[...], 0.0)
    agg2_term = lax.dot_general(agg2_t[...], v2[...], (((0,), (0,)), ((), ())),
                                preferred_element_type=jnp.float32)
    h = jnp.maximum(
        jnp.dot(nn2, v1[...], preferred_element_type=jnp.float32)
        + agg2_term + b3[...], 0.0)
    hh_o[...] = jnp.tanh(
        jnp.dot(h, wg[...], preferred_element_type=jnp.float32) + bg[...])


def _node_fin(nn1, agg_t, agg2_t, u1, u2, bu, w, b2, v1, v2, b3, wg, bg):
    n = nn1.shape[0]
    blk = _FIN_BLK
    grid = n // blk
    full = lambda s: pl.BlockSpec(s, lambda i: (0,) * len(s))
    return pl.pallas_call(
        _node_fin_body,
        grid=(grid,),
        in_specs=[
            pl.BlockSpec((blk, H), lambda i: (i, 0)),
            pl.BlockSpec((H, blk), lambda i: (0, i)),
            pl.BlockSpec((H, blk), lambda i: (0, i)),
            full((H, H)), full((H, H)), full((1, H)),
            full((H, H)), full((1, H)),
            full((H, H)), full((H, H)), full((1, H)),
            full((H, H)), full((1, H)),
        ],
        out_specs=pl.BlockSpec((blk, H), lambda i: (i, 0)),
        out_shape=jax.ShapeDtypeStruct((n, H), jnp.float32),
    )(nn1, agg_t, agg2_t, u1, u2, bu.reshape(1, H), w, b2.reshape(1, H),
      v1, v2, b3.reshape(1, H), wg, bg.reshape(1, H))


def _branch(nf, ef, src, dst, gid, p):
    n = nf.shape[0]
    g = 1024

    wu = p['l1_upd_e'][0]
    w1, w2, w3 = wu[:H], wu[H:2 * H], wu[2 * H:]

    nn1, a_tab, b_tab = _node_pre(
        nf, p['l1_n2n'][0], p['l1_n2n'][1], p['l1_left'][0], p['l1_left'][1],
        w1, p['l1_right'][0], p['l1_right'][1], w2)

    ef8 = jnp.pad(ef, ((0, 0), (0, 2)))
    we2n8 = jnp.pad(p['l1_e2n'][0], ((0, 2), (0, 0)))
    we2e8 = jnp.pad(p['l1_e2e'][0], ((0, 2), (0, 0)))

    asrc, bdst = _sc_gather2(a_tab, b_tab, src, dst)
    e2n_t = _e2n_t(ef8, we2n8, p['l1_e2n'][1])
    e2n2_t = _edge2_t(ef8, asrc, bdst, we2e8, p['l1_e2e'][1], w3,
                      p['l1_upd_e'][1], p['l2_e2n'][0], p['l2_e2n'][1])

    agg_t = _sc_segsum_t(e2n_t, dst)
    agg2_t = _sc_segsum_t(e2n2_t, dst)

    wun = p['l1_upd_n'][0]
    wun2 = p['l2_upd_n'][0]
    pad = _N_PAD - n
    hh = _node_fin(
        jnp.pad(nn1, ((0, pad), (0, 0))),
        jnp.pad(agg_t, ((0, 0), (0, pad))),
        jnp.pad(agg2_t, ((0, 0), (0, pad))),
        wun[:H], wun[H:], p['l1_upd_n'][1],
        p['l2_n2n'][0], p['l2_n2n'][1], wun2[:H], wun2[H:], p['l2_upd_n'][1],
        p['n2g'][0], p['n2g'][1])[:n]

    mu = jnp.mean(hh, axis=0)
    var = jnp.mean(hh * hh, axis=0) - mu * mu
    gamma, beta = p['bn1']
    scale = gamma * jax.lax.rsqrt(var + 1e-5)
    shift = beta - scale * mu

    seg = jax.ops.segment_sum(hh, gid, num_segments=g)
    cnt = (jnp.searchsorted(gid, jnp.arange(1, g + 1, dtype=jnp.int32),
                            side='left')
           - jnp.searchsorted(gid, jnp.arange(g, dtype=jnp.int32),
                              side='left')).astype(jnp.float32)
    gfeat = seg * scale + cnt[:, None] * shift

    wp, bp = p['pred']
    return gfeat @ wp + bp


def kernel(node_feats1, edge_feats1, node_feats2, edge_feats2, edge_index1,
           graph_ids1, edge_index2, graph_ids2, params):
    s1 = _branch(node_feats1, edge_feats1, edge_index1[0], edge_index1[1],
                 graph_ids1, params)
    s2 = _branch(node_feats2, edge_feats2, edge_index2[0], edge_index2[1],
                 graph_ids2, params)
    diff = s1 - s2
    wf, bf = params['fc']
    x = diff @ wf + bf
    g2, b2 = params['bn2']
    mu = jnp.mean(x, axis=0)
    var = jnp.mean((x - mu) ** 2, axis=0)
    x = jnp.maximum(g2 * (x - mu) * jax.lax.rsqrt(var + 1e-5) + b2, 0.0)
    wo, bo = params['out']
    return jnp.squeeze(x @ wo + bo, axis=-1)


# speedup vs baseline: 1.3732x; 1.3732x over previous
import jax
import jax.numpy as jnp
from jax import lax
from jax.experimental import pallas as pl
from jax.experimental.pallas import tpu as pltpu
from jax.experimental.pallas import tpu_sc as plsc

H = 128
N_NODES = 50000
E_EDGES = 800000
_NW = 32
_C = 3200
_NCHUNK = E_EDGES // _C
_GC = 128
_GNCH = E_EDGES // _GC
_GQ, _GR = divmod(_GNCH, _NW)
_NODE_BLK = 2000
_FIN_BLK = 2048
_N_PAD = 51200
_EDGE_BLK = 3200


def _segsum_body(vt_hbm, dst_hbm, out_hbm,
                 d_a, v0_a, v1_a, d_b, v0_b, v1_b,
                 acc0, acc1, sem_a, sem_b):
    wid = lax.axis_index("s") * 2 + lax.axis_index("c")

    def start(chunk, f0, bufs, sem):
        d_buf, v0_buf, v1_buf = bufs
        pltpu.async_copy(dst_hbm.at[pl.ds(chunk * _C, _C)], d_buf, sem)
        pltpu.async_copy(vt_hbm.at[f0, pl.ds(chunk * _C, _C)], v0_buf, sem)
        pltpu.async_copy(vt_hbm.at[f0 + 1, pl.ds(chunk * _C, _C)], v1_buf, sem)

    def wait(bufs, sem):
        d_buf, v0_buf, v1_buf = bufs
        pltpu.make_async_copy(dst_hbm.at[pl.ds(0, _C)], d_buf, sem).wait()
        pltpu.make_async_copy(vt_hbm.at[0, pl.ds(0, _C)], v0_buf, sem).wait()
        pltpu.make_async_copy(vt_hbm.at[0, pl.ds(0, _C)], v1_buf, sem).wait()

    def consume(bufs):
        d_buf, v0_buf, v1_buf = bufs

        def inner(j, _):
            d = d_buf[pl.ds(j * 16, 16)]
            plsc.addupdate_scatter(acc0, [d], v0_buf[pl.ds(j * 16, 16)])
            plsc.addupdate_scatter(acc1, [d], v1_buf[pl.ds(j * 16, 16)])
            return 0

        lax.fori_loop(0, _C // 16, inner, 0)

    bufs_a = (d_a, v0_a, v1_a)
    bufs_b = (d_b, v0_b, v1_b)
    zeros16 = jnp.zeros((16,), jnp.float32)

    for p in range(2):
        f0 = (p * _NW + wid) * 2

        def zero(i, _):
            acc0[pl.ds(i * 16, 16)] = zeros16
            acc1[pl.ds(i * 16, 16)] = zeros16
            return 0

        lax.fori_loop(0, N_NODES // 16, zero, 0)

        start(0, f0, bufs_a, sem_a)

        def body2(h, _):
            c0 = h * 2
            start(c0 + 1, f0, bufs_b, sem_b)
            wait(bufs_a, sem_a)
            consume(bufs_a)

            @pl.when(c0 + 2 < _NCHUNK)
            def _():
                start(c0 + 2, f0, bufs_a, sem_a)

            wait(bufs_b, sem_b)
            consume(bufs_b)
            return 0

        lax.fori_loop(0, _NCHUNK // 2, body2, 0)

        pltpu.sync_copy(acc0, out_hbm.at[f0])
        pltpu.sync_copy(acc1, out_hbm.at[f0 + 1])


def _sc_segsum_t(vt, dst):
    mesh = plsc.VectorSubcoreMesh(core_axis_name="c", subcore_axis_name="s")
    return pl.kernel(
        _segsum_body,
        mesh=mesh,
        compiler_params=pltpu.CompilerParams(needs_layout_passes=False),
        out_type=jax.ShapeDtypeStruct((H, N_NODES), jnp.float32),
        scratch_types=[
            pltpu.VMEM((_C,), jnp.int32),
            pltpu.VMEM((_C,), jnp.float32),
            pltpu.VMEM((_C,), jnp.float32),
            pltpu.VMEM((_C,), jnp.int32),
            pltpu.VMEM((_C,), jnp.float32),
            pltpu.VMEM((_C,), jnp.float32),
            pltpu.VMEM((N_NODES,), jnp.float32),
            pltpu.VMEM((N_NODES,), jnp.float32),
            pltpu.SemaphoreType.DMA,
            pltpu.SemaphoreType.DMA,
        ],
    )(vt, dst)


def _gather2_body(a_hbm, b_hbm, src2_hbm, dst2_hbm, oa_hbm, ob_hbm,
                  ia, ib, arows, brows, sem):
    wid = lax.axis_index("s") * 2 + lax.axis_index("c")
    lo = wid * _GQ + jnp.minimum(wid, _GR)
    hi = (wid + 1) * _GQ + jnp.minimum(wid + 1, _GR)

    def chunk(c, _):
        pltpu.sync_copy(src2_hbm.at[c], ia)
        pltpu.sync_copy(dst2_hbm.at[c], ib)
        cp_a = pltpu.async_copy(a_hbm.at[ia], arows, sem)
        cp_b = pltpu.async_copy(b_hbm.at[ib], brows, sem)
        cp_a.wait()
        cp_b.wait()
        pltpu.sync_copy(arows, oa_hbm.at[pl.ds(c * _GC, _GC)])
        pltpu.sync_copy(brows, ob_hbm.at[pl.ds(c * _GC, _GC)])
        return 0

    lax.fori_loop(lo, hi, chunk, 0)


def _sc_gather2(a_tab, b_tab, src, dst):
    mesh = plsc.VectorSubcoreMesh(core_axis_name="c", subcore_axis_name="s")
    src2 = src.reshape(_GNCH, _GC)
    dst2 = dst.reshape(_GNCH, _GC)
    return pl.kernel(
        _gather2_body,
        mesh=mesh,
        compiler_params=pltpu.CompilerParams(needs_layout_passes=False),
        out_type=(jax.ShapeDtypeStruct((E_EDGES, H), jnp.float32),
                  jax.ShapeDtypeStruct((E_EDGES, H), jnp.float32)),
        scratch_types=[
            pltpu.VMEM((_GC,), jnp.int32),
            pltpu.VMEM((_GC,), jnp.int32),
            pltpu.VMEM((_GC, H), jnp.float32),
            pltpu.VMEM((_GC, H), jnp.float32),
            pltpu.SemaphoreType.DMA,
        ],
    )(a_tab, b_tab, src2, dst2)


def _node_pre_body(nf, wn, bn, wl, bl, w1, wr, br, w2, nn1_o, a_o, b_o):
    x = nf[...]
    nn1_o[...] = jnp.maximum(
        jnp.dot(x, wn[...], preferred_element_type=jnp.float32) + bn[...], 0.0)
    left = jnp.maximum(
        jnp.dot(x, wl[...], preferred_element_type=jnp.float32) + bl[...], 0.0)
    a_o[...] = jnp.dot(left, w1[...], preferred_element_type=jnp.float32)
    right = jnp.maximum(
        jnp.dot(x, wr[...], preferred_element_type=jnp.float32) + br[...], 0.0)
    b_o[...] = jnp.dot(right, w2[...], preferred_element_type=jnp.float32)


def _node_pre(nf, wn, bn, wl, bl, w1, wr, br, w2):
    n = nf.shape[0]
    blk = _NODE_BLK
    full = lambda s: pl.BlockSpec(s, lambda i: (0,) * len(s))
    return pl.pallas_call(
        _node_pre_body,
        grid=(n // blk,),
        in_specs=[
            pl.BlockSpec((blk, nf.shape[1]), lambda i: (i, 0)),
            full(wn.shape), full((1, H)), full(wl.shape), full((1, H)),
            full(w1.shape), full(wr.shape), full((1, H)), full(w2.shape),
        ],
        out_specs=[pl.BlockSpec((blk, H), lambda i: (i, 0))] * 3,
        out_shape=[jax.ShapeDtypeStruct((n, H), jnp.float32)] * 3,
    )(nf, wn, bn.reshape(1, H), wl, bl.reshape(1, H), w1, wr,
      br.reshape(1, H), w2)


def _e2nT_body(ef, we2n, bcol, out_t):
    out_t[...] = jnp.maximum(
        lax.dot_general(we2n[...], ef[...], (((0,), (1,)), ((), ())),
                        preferred_element_type=jnp.float32) + bcol[...], 0.0)


def _e2n_t(ef8, we2n8, be2n):
    blk = _EDGE_BLK
    full = lambda s: pl.BlockSpec(s, lambda i: (0,) * len(s))
    return pl.pallas_call(
        _e2nT_body,
        grid=(E_EDGES // blk,),
        in_specs=[
            pl.BlockSpec((blk, 8), lambda i: (i, 0)),
            full((8, H)), full((H, 1)),
        ],
        out_specs=pl.BlockSpec((H, blk), lambda i: (0, i)),
        out_shape=jax.ShapeDtypeStruct((H, E_EDGES), jnp.float32),
    )(ef8, we2n8, be2n.reshape(H, 1))


def _edge2_body(ef, asrc, bdst, we2e, be2e, w3, bu, wl2, bl2col, out_t):
    x = ef[...]
    e2e = jnp.maximum(
        jnp.dot(x, we2e[...], preferred_element_type=jnp.float32) + be2e[...], 0.0)
    t = jnp.dot(e2e, w3[...], preferred_element_type=jnp.float32)
    new_e = jnp.maximum(asrc[...] + bdst[...] + t + bu[...], 0.0)
    out_t[...] = jnp.maximum(
        lax.dot_general(wl2[...], new_e, (((0,), (1,)), ((), ())),
                        preferred_element_type=jnp.float32) + bl2col[...], 0.0)


def _edge2_t(ef8, asrc, bdst, we2e8, be2e, w3, bu, wl2, bl2):
    blk = _EDGE_BLK
    full = lambda s: pl.BlockSpec(s, lambda i: (0,) * len(s))
    return pl.pallas_call(
        _edge2_body,
        grid=(E_EDGES // blk,),
        in_specs=[
            pl.BlockSpec((blk, 8), lambda i: (i, 0)),
            pl.BlockSpec((blk, H), lambda i: (i, 0)),
            pl.BlockSpec((blk, H), lambda i: (i, 0)),
            full((8, H)), full((1, H)), full((H, H)), full((1, H)),
            full((H, H)), full((H, 1)),
        ],
        out_specs=pl.BlockSpec((H, blk), lambda i: (0, i)),
        out_shape=jax.ShapeDtypeStruct((H, E_EDGES), jnp.float32),
    )(ef8, asrc, bdst, we2e8, be2e.reshape(1, H), w3, bu.reshape(1, H),
      wl2, bl2.reshape(H, 1))


def _node_fin_body(nn1, agg_t, agg2_t, u1, u2, bu, w, b2, v1, v2, b3, wg, bg,
                   hh_o):
    agg_term = lax.dot_general(agg_t[...], u2[...], (((0,), (0,)), ((), ())),
                               preferred_element_type=jnp.float32)
    new_n = jnp.maximum(
        jnp.dot(nn1[...], u1[...], preferred_element_type=jnp.float32)
        + agg_term + bu[...], 0.0)
    nn2 = jnp.maximum(
        jnp.dot(new_n, w[...], preferred_element_type=jnp.float32) + b2[...], 0.0)
    agg2_term = lax.dot_general(agg2_t[...], v2[...], (((0,), (0,)), ((), ())),
                                preferred_element_type=jnp.float32)
    h = jnp.maximum(
        jnp.dot(nn2, v1[...], preferred_element_type=jnp.float32)
        + agg2_term + b3[...], 0.0)
    hh_o[...] = jnp.tanh(
        jnp.dot(h, wg[...], preferred_element_type=jnp.float32) + bg[...])


def _node_fin(nn1, agg_t, agg2_t, u1, u2, bu, w, b2, v1, v2, b3, wg, bg):
    n = nn1.shape[0]
    blk = _FIN_BLK
    grid = n // blk
    full = lambda s: pl.BlockSpec(s, lambda i: (0,) * len(s))
    return pl.pallas_call(
        _node_fin_body,
        grid=(grid,),
        in_specs=[
            pl.BlockSpec((blk, H), lambda i: (i, 0)),
            pl.BlockSpec((H, blk), lambda i: (0, i)),
            pl.BlockSpec((H, blk), lambda i: (0, i)),
            full((H, H)), full((H, H)), full((1, H)),
            full((H, H)), full((1, H)),
            full((H, H)), full((H, H)), full((1, H)),
            full((H, H)), full((1, H)),
        ],
        out_specs=pl.BlockSpec((blk, H), lambda i: (i, 0)),
        out_shape=jax.ShapeDtypeStruct((n, H), jnp.float32),
    )(nn1, agg_t, agg2_t, u1, u2, bu.reshape(1, H), w, b2.reshape(1, H),
      v1, v2, b3.reshape(1, H), wg, bg.reshape(1, H))


def _branch(nf, ef, src, dst, gid, p):
    n = nf.shape[0]
    g = 1024

    wu = p['l1_upd_e'][0]
    w1, w2, w3 = wu[:H], wu[H:2 * H], wu[2 * H:]

    nn1, a_tab, b_tab = _node_pre(
        nf, p['l1_n2n'][0], p['l1_n2n'][1], p['l1_left'][0], p['l1_left'][1],
        w1, p['l1_right'][0], p['l1_right'][1], w2)

    ef8 = jnp.pad(ef, ((0, 0), (0, 2)))
    we2n8 = jnp.pad(p['l1_e2n'][0], ((0, 2), (0, 0)))
    we2e8 = jnp.pad(p['l1_e2e'][0], ((0, 2), (0, 0)))

    asrc, bdst = _sc_gather2(a_tab, b_tab, src, dst)
    e2n_t = _e2n_t(ef8, we2n8, p['l1_e2n'][1])
    e2n2_t = _edge2_t(ef8, asrc, bdst, we2e8, p['l1_e2e'][1], w3,
                      p['l1_upd_e'][1], p['l2_e2n'][0], p['l2_e2n'][1])

    agg_t = _sc_segsum_t(e2n_t, dst)
    agg2_t = _sc_segsum_t(e2n2_t, dst)

    wun = p['l1_upd_n'][0]
    wun2 = p['l2_upd_n'][0]
    pad = _N_PAD - n
    hh = _node_fin(
        jnp.pad(nn1, ((0, pad), (0, 0))),
        jnp.pad(agg_t, ((0, 0), (0, pad))),
        jnp.pad(agg2_t, ((0, 0), (0, pad))),
        wun[:H], wun[H:], p['l1_upd_n'][1],
        p['l2_n2n'][0], p['l2_n2n'][1], wun2[:H], wun2[H:], p['l2_upd_n'][1],
        p['n2g'][0], p['n2g'][1])[:n]

    mu = jnp.mean(hh, axis=0)
    var = jnp.mean(hh * hh, axis=0) - mu * mu
    gamma, beta = p['bn1']
    scale = gamma * jax.lax.rsqrt(var + 1e-5)
    shift = beta - scale * mu

    seg = jax.ops.segment_sum(hh, gid, num_segments=g)
    cnt = (jnp.searchsorted(gid, jnp.arange(1, g + 1, dtype=jnp.int32),
                            side='left')
           - jnp.searchsorted(gid, jnp.arange(g, dtype=jnp.int32),
                              side='left')).astype(jnp.float32)
    gfeat = seg * scale + cnt[:, None] * shift

    wp, bp = p['pred']
    return gfeat @ wp + bp


def kernel(node_feats1, edge_feats1, node_feats2, edge_feats2, edge_index1,
           graph_ids1, edge_index2, graph_ids2, params):
    s1 = _branch(node_feats1, edge_feats1, edge_index1[0], edge_index1[1],
                 graph_ids1, params)
    s2 = _branch(node_feats2, edge_feats2, edge_index2[0], edge_index2[1],
                 graph_ids2, params)
    diff = s1 - s2
    wf, bf = params['fc']
    x = diff @ wf + bf
    g2, b2 = params['bn2']
    mu = jnp.mean(x, axis=0)
    var = jnp.mean((x - mu) ** 2, axis=0)
    x = jnp.maximum(g2 * (x - mu) * jax.lax.rsqrt(var + 1e-5) + b2, 0.0)
    wo, bo = params['out']
    return jnp.squeeze(x @ wo + bo, axis=-1)
